# baseline (speedup 1.0000x reference)
import functools

import jax
import jax.numpy as jnp
from jax import lax
from jax.experimental import pallas as pl
from jax.experimental.pallas import tpu as pltpu
from jax.experimental.pallas import tpu_sc as plsc

_N = 10000
_D = 128
_E = 320000
_NC = 2
_NS = 16
_NW = _NC * _NS
_CHUNK = 128
_NPAD = 10240
_ROWS_PER_SUB = _NPAD // _NS
_CHUNKS_PER_W = 80
_EDGES_PER_W = _CHUNKS_PER_W * _CHUNK
_EP = _NW * _EDGES_PER_W
_NGROUPS = _CHUNKS_PER_W // 4
_BLK = 256


def _sc_degree(dst_pad):
    mesh = plsc.VectorSubcoreMesh(core_axis_name="c", subcore_axis_name="s")

    @functools.partial(
        pl.kernel,
        out_type=jax.ShapeDtypeStruct((_NC, _NPAD), jnp.float32),
        mesh=mesh,
        scratch_types=[
            pltpu.VMEM((_CHUNK,), jnp.int32),
            pltpu.VMEM((_CHUNK,), jnp.int32),
            pltpu.VMEM((_CHUNK,), jnp.float32),
            pltpu.VMEM((_ROWS_PER_SUB,), jnp.float32),
            pltpu.VMEM_SHARED((_NPAD,), jnp.float32),
            pltpu.SemaphoreType.DMA((2,)),
        ],
    )
    def k(dst_hbm, deg_hbm, iA, iB, ones_v, zeros_v, deg_sh, isem):
        c = lax.axis_index("c")
        s = lax.axis_index("s")
        w = c * _NS + s
        base0 = w * _EDGES_PER_W

        def fill_ones(i, _):
            ones_v[pl.ds(i * 16, 16)] = jnp.ones((16,), jnp.float32)
            return 0

        lax.fori_loop(0, _CHUNK // 16, fill_ones, 0)

        def fill_zeros(i, _):
            zeros_v[pl.ds(i * 16, 16)] = jnp.zeros((16,), jnp.float32)
            return 0

        lax.fori_loop(0, _ROWS_PER_SUB // 16, fill_zeros, 0)
        pltpu.sync_copy(zeros_v, deg_sh.at[pl.ds(s * _ROWS_PER_SUB, _ROWS_PER_SUB)])
        plsc.subcore_barrier()

        pltpu.async_copy(dst_hbm.at[pl.ds(base0, _CHUNK)], iA, isem.at[0])
        pltpu.async_copy(dst_hbm.at[pl.ds(base0 + _CHUNK, _CHUNK)], iB,
                         isem.at[1])

        def body(i, _):
            for par, iv in ((0, iA), (1, iB)):
                ci = 2 * i + par
                pltpu.make_async_copy(
                    dst_hbm.at[pl.ds(0, _CHUNK)], iv, isem.at[par]).wait()
                pltpu.sync_copy(ones_v, deg_sh.at[iv], add=True)

                @pl.when(ci < _CHUNKS_PER_W - 2)
                def _():
                    pltpu.async_copy(
                        dst_hbm.at[pl.ds(base0 + (ci + 2) * _CHUNK, _CHUNK)],
                        iv, isem.at[par])
            return 0

        lax.fori_loop(0, _CHUNKS_PER_W // 2, body, 0)
        plsc.subcore_barrier()
        pltpu.sync_copy(
            deg_sh.at[pl.ds(s * _ROWS_PER_SUB, _ROWS_PER_SUB)],
            deg_hbm.at[c, pl.ds(s * _ROWS_PER_SUB, _ROWS_PER_SUB)],
        )

    return k(dst_pad)


def _sc_gather_scatter(g, src_pad, dst_pad):
    mesh = plsc.VectorSubcoreMesh(core_axis_name="c", subcore_axis_name="s")

    @functools.partial(
        pl.kernel,
        out_type=jax.ShapeDtypeStruct((_NC, _NPAD, _D), jnp.float32),
        mesh=mesh,
        scratch_types=[
            pltpu.VMEM((_CHUNK,), jnp.int32),
            pltpu.VMEM((_CHUNK,), jnp.int32),
            pltpu.VMEM((_CHUNK,), jnp.int32),
            pltpu.VMEM((_CHUNK,), jnp.int32),
            pltpu.VMEM((2, _CHUNK, _D), jnp.float32),
            pltpu.VMEM_SHARED((_NPAD, _D), jnp.float32),
            pltpu.SemaphoreType.DMA((2,)),
            pltpu.SemaphoreType.DMA((2,)),
        ],
    )
    def k(g_hbm, src_hbm, dst_hbm, acc_hbm, sA, sB, dA, dB, rows, acc_sh,
          gsem, isem):
        c = lax.axis_index("c")
        s = lax.axis_index("s")
        w = c * _NS + s
        base0 = w * _EDGES_PER_W

        @pl.when(c == 0)
        def _():
            pltpu.sync_copy(
                g_hbm.at[pl.ds(s * _ROWS_PER_SUB, _ROWS_PER_SUB)],
                acc_sh.at[pl.ds(s * _ROWS_PER_SUB, _ROWS_PER_SUB)],
            )

        @pl.when(c == 1)
        def _():
            def zero_row(r, _):
                for j in range(_D // 16):
                    rows[0, r, pl.ds(j * 16, 16)] = jnp.zeros(
                        (16,), jnp.float32)
                return 0

            lax.fori_loop(0, _CHUNK, zero_row, 0)
            for t in range(_ROWS_PER_SUB // _CHUNK):
                pltpu.sync_copy(
                    rows.at[0],
                    acc_sh.at[pl.ds(s * _ROWS_PER_SUB + t * _CHUNK, _CHUNK)],
                )

        plsc.subcore_barrier()

        pltpu.async_copy(src_hbm.at[pl.ds(base0, _CHUNK)], sA, isem.at[0])
        pltpu.async_copy(dst_hbm.at[pl.ds(base0, _CHUNK)], dA, isem.at[0])
        pltpu.async_copy(src_hbm.at[pl.ds(base0 + _CHUNK, _CHUNK)], sB,
                         isem.at[1])
        pltpu.async_copy(dst_hbm.at[pl.ds(base0 + _CHUNK, _CHUNK)], dB,
                         isem.at[1])
        pltpu.make_async_copy(
            src_hbm.at[pl.ds(0, _CHUNK)], sA, isem.at[0]).wait()
        pltpu.make_async_copy(
            dst_hbm.at[pl.ds(0, _CHUNK)], dA, isem.at[0]).wait()
        pltpu.async_copy(g_hbm.at[sA], rows.at[0], gsem.at[0])

        def body(i, _):
            for par, si, di, so, do in ((0, sA, dA, sB, dB),
                                        (1, sB, dB, sA, dA)):
                ci = 2 * i + par

                @pl.when(ci < _CHUNKS_PER_W - 1)
                def _():
                    pltpu.make_async_copy(
                        src_hbm.at[pl.ds(0, _CHUNK)], so,
                        isem.at[1 - par]).wait()
                    pltpu.make_async_copy(
                        dst_hbm.at[pl.ds(0, _CHUNK)], do,
                        isem.at[1 - par]).wait()
                    pltpu.async_copy(g_hbm.at[so], rows.at[1 - par],
                                     gsem.at[1 - par])

                pltpu.make_async_copy(
                    g_hbm.at[pl.ds(0, _CHUNK)], rows.at[par],
                    gsem.at[par]).wait()

                @pl.when(ci < _CHUNKS_PER_W - 2)
                def _():
                    pltpu.async_copy(
                        src_hbm.at[pl.ds(base0 + (ci + 2) * _CHUNK, _CHUNK)],
                        si, isem.at[par])

                pltpu.sync_copy(rows.at[par], acc_sh.at[di], add=True)

                @pl.when(ci < _CHUNKS_PER_W - 2)
                def _():
                    pltpu.async_copy(
                        dst_hbm.at[pl.ds(base0 + (ci + 2) * _CHUNK, _CHUNK)],
                        di, isem.at[par])
            return 0

        lax.fori_loop(0, _CHUNKS_PER_W // 2, body, 0)
        plsc.subcore_barrier()
        pltpu.sync_copy(
            acc_sh.at[pl.ds(s * _ROWS_PER_SUB, _ROWS_PER_SUB)],
            acc_hbm.at[c, pl.ds(s * _ROWS_PER_SUB, _ROWS_PER_SUB)],
        )

    return k(g, src_pad, dst_pad)


def _tc_pre(z_pad, W1, b1r, Wg, deg):
    def body(z_ref, w1_ref, b1_ref, wg_ref, deg_ref, g_ref):
        i = pl.program_id(0)
        h = jnp.maximum(
            jnp.dot(z_ref[...], w1_ref[...], preferred_element_type=jnp.float32)
            + b1_ref[...],
            0.0,
        )
        h2 = jnp.dot(h, wg_ref[...], preferred_element_type=jnp.float32)
        dsum = (
            deg_ref[0, pl.ds(i * _BLK, _BLK)]
            + deg_ref[1, pl.ds(i * _BLK, _BLK)]
            + 1.0
        )
        dinv = lax.rsqrt(dsum)
        g_ref[...] = h2 * dinv[:, None]

    return pl.pallas_call(
        body,
        grid=(_NPAD // _BLK,),
        in_specs=[
            pl.BlockSpec((_BLK, _D), lambda i: (i, 0)),
            pl.BlockSpec((_D, _D), lambda i: (0, 0)),
            pl.BlockSpec((1, _D), lambda i: (0, 0)),
            pl.BlockSpec((_D, _D), lambda i: (0, 0)),
            pl.BlockSpec((_NC, _NPAD), lambda i: (0, 0)),
        ],
        out_specs=pl.BlockSpec((_BLK, _D), lambda i: (i, 0)),
        out_shape=jax.ShapeDtypeStruct((_NPAD, _D), jnp.float32),
    )(z_pad, W1, b1r, Wg, deg)


def _tc_post(acc, deg, bgr, W2, b2r):
    def body(acc_ref, deg_ref, bg_ref, w2_ref, b2_ref, out_ref):
        i = pl.program_id(0)
        dsum = (
            deg_ref[0, pl.ds(i * _BLK, _BLK)]
            + deg_ref[1, pl.ds(i * _BLK, _BLK)]
            + 1.0
        )
        dinv = lax.rsqrt(dsum)
        x = (acc_ref[0] + acc_ref[1]) * dinv[:, None]
        h3 = jnp.maximum(x + bg_ref[...], 0.0)
        out_ref[...] = (
            jnp.dot(h3, w2_ref[...], preferred_element_type=jnp.float32)
            + b2_ref[...]
        )

    return pl.pallas_call(
        body,
        grid=(_NPAD // _BLK,),
        in_specs=[
            pl.BlockSpec((_NC, _BLK, _D), lambda i: (0, i, 0)),
            pl.BlockSpec((_NC, _NPAD), lambda i: (0, 0)),
            pl.BlockSpec((1, _D), lambda i: (0, 0)),
            pl.BlockSpec((_D, _D), lambda i: (0, 0)),
            pl.BlockSpec((1, _D), lambda i: (0, 0)),
        ],
        out_specs=pl.BlockSpec((_BLK, _D), lambda i: (i, 0)),
        out_shape=jax.ShapeDtypeStruct((_NPAD, _D), jnp.float32),
    )(acc, deg, bgr, W2, b2r)


def kernel(z, edge_index, W1, b1, Wg, bg, W2, b2):
    src = edge_index[0]
    dst = edge_index[1]
    pad_e = _EP - _E
    pad_iota = jnp.arange(pad_e, dtype=jnp.int32)
    src_p = jnp.concatenate([src, pad_iota % _N])
    dst_p = jnp.concatenate([dst, _N + pad_iota % (_NPAD - _N)])
    z_pad = jnp.pad(z, ((0, _NPAD - _N), (0, 0)))

    deg = _sc_degree(dst_p)
    g = _tc_pre(z_pad, W1, b1.reshape(1, _D), Wg, deg)
    acc = _sc_gather_scatter(g, src_p, dst_p)
    out = _tc_post(acc, deg, bg.reshape(1, _D), W2, b2.reshape(1, _D))
    return out[:_N]


# speedup vs baseline: 1.1037x; 1.1037x over previous
import functools

import jax
import jax.numpy as jnp
from jax import lax
from jax.experimental import pallas as pl
from jax.experimental.pallas import tpu as pltpu
from jax.experimental.pallas import tpu_sc as plsc

_N = 10000
_D = 128
_E = 320000
_NC = 2
_NS = 16
_NW = _NC * _NS
_CHUNK = 128
_NPAD = 10240
_ROWS_PER_SUB = _NPAD // _NS
_CHUNKS_PER_W = 80
_EDGES_PER_W = _CHUNKS_PER_W * _CHUNK
_EP = _NW * _EDGES_PER_W
_NGROUPS = _CHUNKS_PER_W // 4
_BLK = 512


def _sc_degree(dst_pad):
    mesh = plsc.VectorSubcoreMesh(core_axis_name="c", subcore_axis_name="s")

    @functools.partial(
        pl.kernel,
        out_type=jax.ShapeDtypeStruct((_NC, _NPAD), jnp.float32),
        mesh=mesh,
        scratch_types=[
            pltpu.VMEM((_CHUNK,), jnp.int32),
            pltpu.VMEM((_CHUNK,), jnp.int32),
            pltpu.VMEM((_CHUNK,), jnp.float32),
            pltpu.VMEM((_ROWS_PER_SUB,), jnp.float32),
            pltpu.VMEM_SHARED((_NPAD,), jnp.float32),
            pltpu.SemaphoreType.DMA((2,)),
        ],
    )
    def k(dst_hbm, deg_hbm, iA, iB, ones_v, zeros_v, deg_sh, isem):
        c = lax.axis_index("c")
        s = lax.axis_index("s")
        w = c * _NS + s
        base0 = w * _EDGES_PER_W

        def fill_ones(i, _):
            ones_v[pl.ds(i * 16, 16)] = jnp.ones((16,), jnp.float32)
            return 0

        lax.fori_loop(0, _CHUNK // 16, fill_ones, 0)

        def fill_zeros(i, _):
            zeros_v[pl.ds(i * 16, 16)] = jnp.zeros((16,), jnp.float32)
            return 0

        lax.fori_loop(0, _ROWS_PER_SUB // 16, fill_zeros, 0)
        pltpu.sync_copy(zeros_v, deg_sh.at[pl.ds(s * _ROWS_PER_SUB, _ROWS_PER_SUB)])
        plsc.subcore_barrier()

        pltpu.async_copy(dst_hbm.at[pl.ds(base0, _CHUNK)], iA, isem.at[0])
        pltpu.async_copy(dst_hbm.at[pl.ds(base0 + _CHUNK, _CHUNK)], iB,
                         isem.at[1])

        def body(i, _):
            for par, iv in ((0, iA), (1, iB)):
                ci = 2 * i + par
                pltpu.make_async_copy(
                    dst_hbm.at[pl.ds(0, _CHUNK)], iv, isem.at[par]).wait()
                pltpu.sync_copy(ones_v, deg_sh.at[iv], add=True)

                @pl.when(ci < _CHUNKS_PER_W - 2)
                def _():
                    pltpu.async_copy(
                        dst_hbm.at[pl.ds(base0 + (ci + 2) * _CHUNK, _CHUNK)],
                        iv, isem.at[par])
            return 0

        lax.fori_loop(0, _CHUNKS_PER_W // 2, body, 0)
        plsc.subcore_barrier()
        pltpu.sync_copy(
            deg_sh.at[pl.ds(s * _ROWS_PER_SUB, _ROWS_PER_SUB)],
            deg_hbm.at[c, pl.ds(s * _ROWS_PER_SUB, _ROWS_PER_SUB)],
        )

    return k(dst_pad)


def _sc_gather_scatter(g, src_pad, dst_pad):
    mesh = plsc.VectorSubcoreMesh(core_axis_name="c", subcore_axis_name="s")

    @functools.partial(
        pl.kernel,
        out_type=jax.ShapeDtypeStruct((_NC, _NPAD, _D), jnp.float32),
        mesh=mesh,
        scratch_types=[
            pltpu.VMEM((_CHUNK,), jnp.int32),
            pltpu.VMEM((_CHUNK,), jnp.int32),
            pltpu.VMEM((_CHUNK,), jnp.int32),
            pltpu.VMEM((_CHUNK,), jnp.int32),
            pltpu.VMEM((2, _CHUNK, _D), jnp.float32),
            pltpu.VMEM_SHARED((_NPAD, _D), jnp.float32),
            pltpu.SemaphoreType.DMA((2,)),
            pltpu.SemaphoreType.DMA((2,)),
        ],
    )
    def k(g_hbm, src_hbm, dst_hbm, acc_hbm, sA, sB, dA, dB, rows, acc_sh,
          gsem, isem):
        c = lax.axis_index("c")
        s = lax.axis_index("s")
        w = c * _NS + s
        base0 = w * _EDGES_PER_W

        @pl.when(c == 0)
        def _():
            pltpu.sync_copy(
                g_hbm.at[pl.ds(s * _ROWS_PER_SUB, _ROWS_PER_SUB)],
                acc_sh.at[pl.ds(s * _ROWS_PER_SUB, _ROWS_PER_SUB)],
            )

        @pl.when(c == 1)
        def _():
            def zero_row(r, _):
                for j in range(_D // 16):
                    rows[0, r, pl.ds(j * 16, 16)] = jnp.zeros(
                        (16,), jnp.float32)
                return 0

            lax.fori_loop(0, _CHUNK, zero_row, 0)
            for t in range(_ROWS_PER_SUB // _CHUNK):
                pltpu.sync_copy(
                    rows.at[0],
                    acc_sh.at[pl.ds(s * _ROWS_PER_SUB + t * _CHUNK, _CHUNK)],
                )

        plsc.subcore_barrier()

        pltpu.async_copy(src_hbm.at[pl.ds(base0, _CHUNK)], sA, isem.at[0])
        pltpu.async_copy(dst_hbm.at[pl.ds(base0, _CHUNK)], dA, isem.at[0])
        pltpu.async_copy(src_hbm.at[pl.ds(base0 + _CHUNK, _CHUNK)], sB,
                         isem.at[1])
        pltpu.async_copy(dst_hbm.at[pl.ds(base0 + _CHUNK, _CHUNK)], dB,
                         isem.at[1])
        pltpu.make_async_copy(
            src_hbm.at[pl.ds(0, _CHUNK)], sA, isem.at[0]).wait()
        pltpu.make_async_copy(
            dst_hbm.at[pl.ds(0, _CHUNK)], dA, isem.at[0]).wait()
        pltpu.async_copy(g_hbm.at[sA], rows.at[0], gsem.at[0])

        def body(i, _):
            for par, si, di, so, do in ((0, sA, dA, sB, dB),
                                        (1, sB, dB, sA, dA)):
                ci = 2 * i + par

                @pl.when(ci < _CHUNKS_PER_W - 1)
                def _():
                    pltpu.make_async_copy(
                        src_hbm.at[pl.ds(0, _CHUNK)], so,
                        isem.at[1 - par]).wait()
                    pltpu.make_async_copy(
                        dst_hbm.at[pl.ds(0, _CHUNK)], do,
                        isem.at[1 - par]).wait()
                    pltpu.async_copy(g_hbm.at[so], rows.at[1 - par],
                                     gsem.at[1 - par])

                pltpu.make_async_copy(
                    g_hbm.at[pl.ds(0, _CHUNK)], rows.at[par],
                    gsem.at[par]).wait()

                @pl.when(ci < _CHUNKS_PER_W - 2)
                def _():
                    pltpu.async_copy(
                        src_hbm.at[pl.ds(base0 + (ci + 2) * _CHUNK, _CHUNK)],
                        si, isem.at[par])

                pltpu.sync_copy(rows.at[par], acc_sh.at[di], add=True)

                @pl.when(ci < _CHUNKS_PER_W - 2)
                def _():
                    pltpu.async_copy(
                        dst_hbm.at[pl.ds(base0 + (ci + 2) * _CHUNK, _CHUNK)],
                        di, isem.at[par])
            return 0

        lax.fori_loop(0, _CHUNKS_PER_W // 2, body, 0)
        plsc.subcore_barrier()
        pltpu.sync_copy(
            acc_sh.at[pl.ds(s * _ROWS_PER_SUB, _ROWS_PER_SUB)],
            acc_hbm.at[c, pl.ds(s * _ROWS_PER_SUB, _ROWS_PER_SUB)],
        )

    return k(g, src_pad, dst_pad)


def _tc_matmuls(z_pad, W1, b1r, Wg):
    def body(z_ref, w1_ref, b1_ref, wg_ref, h2_ref):
        h = jnp.maximum(
            jnp.dot(z_ref[...], w1_ref[...], preferred_element_type=jnp.float32)
            + b1_ref[...],
            0.0,
        )
        h2_ref[...] = jnp.dot(h, wg_ref[...],
                              preferred_element_type=jnp.float32)

    return pl.pallas_call(
        body,
        grid=(_NPAD // _BLK,),
        in_specs=[
            pl.BlockSpec((_BLK, _D), lambda i: (i, 0)),
            pl.BlockSpec((_D, _D), lambda i: (0, 0)),
            pl.BlockSpec((1, _D), lambda i: (0, 0)),
            pl.BlockSpec((_D, _D), lambda i: (0, 0)),
        ],
        out_specs=pl.BlockSpec((_BLK, _D), lambda i: (i, 0)),
        out_shape=jax.ShapeDtypeStruct((_NPAD, _D), jnp.float32),
    )(z_pad, W1, b1r, Wg)


def _tc_scale(h2, deg):
    def body(h2_ref, deg_ref, g_ref):
        i = pl.program_id(0)
        dsum = (
            deg_ref[0, pl.ds(i * _BLK, _BLK)]
            + deg_ref[1, pl.ds(i * _BLK, _BLK)]
            + 1.0
        )
        dinv = lax.rsqrt(dsum)
        g_ref[...] = h2_ref[...] * dinv[:, None]

    return pl.pallas_call(
        body,
        grid=(_NPAD // _BLK,),
        in_specs=[
            pl.BlockSpec((_BLK, _D), lambda i: (i, 0)),
            pl.BlockSpec((_NC, _NPAD), lambda i: (0, 0)),
        ],
        out_specs=pl.BlockSpec((_BLK, _D), lambda i: (i, 0)),
        out_shape=jax.ShapeDtypeStruct((_NPAD, _D), jnp.float32),
    )(h2, deg)


def _tc_post(acc, deg, bgr, W2, b2r):
    def body(acc_ref, deg_ref, bg_ref, w2_ref, b2_ref, out_ref):
        i = pl.program_id(0)
        dsum = (
            deg_ref[0, pl.ds(i * _BLK, _BLK)]
            + deg_ref[1, pl.ds(i * _BLK, _BLK)]
            + 1.0
        )
        dinv = lax.rsqrt(dsum)
        x = (acc_ref[0] + acc_ref[1]) * dinv[:, None]
        h3 = jnp.maximum(x + bg_ref[...], 0.0)
        out_ref[...] = (
            jnp.dot(h3, w2_ref[...], preferred_element_type=jnp.float32)
            + b2_ref[...]
        )

    return pl.pallas_call(
        body,
        grid=(_NPAD // _BLK,),
        in_specs=[
            pl.BlockSpec((_NC, _BLK, _D), lambda i: (0, i, 0)),
            pl.BlockSpec((_NC, _NPAD), lambda i: (0, 0)),
            pl.BlockSpec((1, _D), lambda i: (0, 0)),
            pl.BlockSpec((_D, _D), lambda i: (0, 0)),
            pl.BlockSpec((1, _D), lambda i: (0, 0)),
        ],
        out_specs=pl.BlockSpec((_BLK, _D), lambda i: (i, 0)),
        out_shape=jax.ShapeDtypeStruct((_NPAD, _D), jnp.float32),
    )(acc, deg, bgr, W2, b2r)


def kernel(z, edge_index, W1, b1, Wg, bg, W2, b2):
    src = edge_index[0]
    dst = edge_index[1]
    pad_e = _EP - _E
    pad_iota = jnp.arange(pad_e, dtype=jnp.int32)
    src_p = jnp.concatenate([src, pad_iota % _N])
    dst_p = jnp.concatenate([dst, _N + pad_iota % (_NPAD - _N)])
    z_pad = jnp.pad(z, ((0, _NPAD - _N), (0, 0)))

    deg = _sc_degree(dst_p)
    h2 = _tc_matmuls(z_pad, W1, b1.reshape(1, _D), Wg)
    g = _tc_scale(h2, deg)
    acc = _sc_gather_scatter(g, src_p, dst_p)
    out = _tc_post(acc, deg, bg.reshape(1, _D), W2, b2.reshape(1, _D))
    return out[:_N]
